# baseline (speedup 1.0000x reference)
import functools

import jax
import jax.numpy as jnp
from jax.experimental import pallas as pl
from jax.experimental.pallas import tpu as pltpu

N = 10000
G = 64
HID = 128
FIL = 128
NG = 50
T = 3
CUT = 10.0
K = 32
NC = 10

SMAX = 256
CH = 8
NCH = K // CH

_LOG2 = 0.6931471805599453
_KC = 128.0
_STEP = CUT / (NG - 1)
_COEFF = -0.5 / (_STEP * _STEP)


def _fiota(shape, dim):
    return jax.lax.broadcasted_iota(jnp.int32, shape, dim).astype(jnp.float32)


def _dot(a, b):
    return jax.lax.dot_general(a, b, (((1,), (0,)), ((), ())),
                               preferred_element_type=jnp.float32)


def _ssp(v):
    return jnp.maximum(v, 0.0) + jnp.log(1.0 + jnp.exp(-jnp.abs(v))) - _LOG2


def _body(posz_ref, sz_ref, emb_ref, w1_ref, b1_ref, w2_ref, b2_ref,
          cfw1_ref, cfw2_ref, cfb2_ref, blkw_ref, blkb_ref,
          o1w_ref, o1b_ref, o2w_ref, o2b_ref,
          e1w_ref, e1b_ref, e2w_ref, e2b_ref,
          out_ref, acc_ref):
    f32 = jnp.float32
    g = pl.program_id(0)

    @pl.when(g == 0)
    def _():
        acc_ref[...] = jnp.zeros_like(acc_ref)

    pg = posz_ref[0]
    sz = sz_ref[0, 0, 0]
    posg = pg[:, 0:3]
    zg = pg[:, 3:4]

    eio = _fiota((SMAX, 100), 1)
    h = _dot((eio == zg).astype(f32), emb_ref[...])

    sq = jnp.sum(posg * posg, axis=1, keepdims=True)
    dpp = jax.lax.dot_general(posg, posg, (((1,), (1,)), ((), ())),
                              preferred_element_type=f32)
    sq_j = jax.lax.dot_general(jnp.ones((SMAX, 1), f32), sq,
                               (((1,), (1,)), ((), ())),
                               preferred_element_type=f32)
    d2 = jnp.maximum(sq + sq_j - 2.0 * dpp, 0.0)

    jio = _fiota((SMAX, SMAX), 1)
    iio = _fiota((SMAX, SMAX), 0)
    mask = (jio != iio) & (jio < sz) & (iio < sz)

    ji = jax.lax.broadcasted_iota(jnp.int32, (SMAX, SMAX), 1)
    key = jnp.maximum(_KC - d2, 0.0)
    ki = jax.lax.bitcast_convert_type(key, jnp.int32)
    ki = (ki & jnp.int32(-256)) | (jnp.int32(SMAX - 1) - ji)
    cur = jnp.where(mask, jax.lax.bitcast_convert_type(ki, f32), 0.0)

    m_l = []
    for _ in range(K):
        m = jnp.max(cur, axis=1, keepdims=True)
        cur = jnp.where(cur == m, 0.0, cur)
        m_l.append(m)
    mm = jnp.concatenate(m_l, axis=1)
    mi = jax.lax.bitcast_convert_type(mm, jnp.int32)
    jm = (jnp.int32(SMAX - 1) - (mi & 255)).astype(f32)
    valid = mm > (_KC - CUT * CUT)
    de = jnp.where(valid, jnp.maximum(_KC - mm, 0.0), 1.0)
    distm = jnp.sqrt(jnp.maximum(de, 1e-12))
    wctm = jnp.where(valid, 0.5 * (jnp.cos(distm * (jnp.pi / CUT)) + 1.0),
                     0.0)
    dist_l = [distm[:, k:k + 1] for k in range(K)]
    idx_l = [jm[:, k:k + 1] for k in range(K)]
    wct_l = [wctm[:, k:k + 1] for k in range(K)]

    offs = _fiota((1, NG), 1) * _STEP
    gs_l, oh_l, wc_l = [], [], []
    for c in range(NCH):
        ks = range(c * CH, (c + 1) * CH)
        gs_l.append(jnp.concatenate(
            [jnp.exp(_COEFF * (dist_l[k] - offs) ** 2) for k in ks],
            axis=0).astype(jnp.bfloat16))
        oh_l.append(jnp.concatenate(
            [(jio == idx_l[k]).astype(jnp.bfloat16) for k in ks], axis=0))
        wc_l.append(jnp.concatenate([wct_l[k] for k in ks], axis=0))

    for t in range(T):
        w1 = w1_ref[t]
        b1 = b1_ref[t]
        w2 = w2_ref[t]
        b2 = b2_ref[t]
        xl = _dot(h, cfw1_ref[t]).astype(jnp.bfloat16)
        agg = jnp.zeros((SMAX, FIL), f32)
        for c in range(NCH):
            f1 = _ssp(_dot(gs_l[c], w1.astype(jnp.bfloat16)) + b1)
            wf = (_dot(f1.astype(jnp.bfloat16), w2.astype(jnp.bfloat16))
                  + b2) * wc_l[c]
            xg = _dot(oh_l[c], xl)
            msg = xg * wf
            for kk in range(CH):
                agg = agg + msg[kk * SMAX:(kk + 1) * SMAX, :]
        hc = _dot(agg, cfw2_ref[t]) + cfb2_ref[t]
        h = h + _dot(_ssp(hc), blkw_ref[t]) + blkb_ref[t]

    h2 = _ssp(_dot(h, o1w_ref[...]) + o1b_ref[...])
    pa = _dot(h2, o2w_ref[...]) + o2b_ref[...]
    rv = (_fiota((SMAX, 1), 0) < sz).astype(f32)
    gsum = jnp.sum(pa * rv)
    gio = jax.lax.broadcasted_iota(jnp.int32, (G, 1), 0)
    acc_ref[...] = acc_ref[...] + jnp.where(gio == g, gsum, 0.0)

    @pl.when(g == G - 1)
    def _():
        accv = acc_ref[...]
        hid = jnp.maximum(accv * e1w_ref[...] + e1b_ref[...], 0.0)
        out_ref[...] = _dot(hid, e2w_ref[...]) + e2b_ref[...]


def _full(shape):
    nd = len(shape)
    return pl.BlockSpec(shape, lambda g, _nd=nd: (0,) * _nd)


@functools.partial(jax.jit, static_argnames=("interpret",))
def _run(posz, szf, emb, mlp_w1, mlp_b1, mlp_w2, mlp_b2, cf_w1, cf_w2, cf_b2,
         blk_w, blk_b, out1_w, out1_b, out2_w, out2_b,
         ext1_w, ext1_b, ext2_w, ext2_b, interpret=False):
    return pl.pallas_call(
        _body,
        grid=(G,),
        in_specs=[
            pl.BlockSpec((1, SMAX, 4), lambda g: (g, 0, 0)),
            pl.BlockSpec((1, 1, 1), lambda g: (g, 0, 0)),
            _full((100, HID)),
            _full((T, NG, FIL)), _full((T, 1, FIL)),
            _full((T, FIL, FIL)), _full((T, 1, FIL)),
            _full((T, HID, FIL)),
            _full((T, FIL, HID)), _full((T, 1, HID)),
            _full((T, HID, HID)), _full((T, 1, HID)),
            _full((HID, HID // 2)), _full((1, HID // 2)),
            _full((HID // 2, 1)), _full((1, 1)),
            _full((1, HID)), _full((1, HID)),
            _full((HID, NC)), _full((1, NC)),
        ],
        out_specs=pl.BlockSpec((G, NC), lambda g: (0, 0)),
        out_shape=jax.ShapeDtypeStruct((G, NC), jnp.float32),
        scratch_shapes=[pltpu.VMEM((G, 1), jnp.float32)],
        interpret=interpret,
    )(posz, szf, emb, mlp_w1, mlp_b1, mlp_w2, mlp_b2, cf_w1, cf_w2, cf_b2,
      blk_w, blk_b, out1_w, out1_b, out2_w, out2_b,
      ext1_w, ext1_b, ext2_w, ext2_b)


def kernel(x, pos, batch, emb, mlp_w1, mlp_b1, mlp_w2, mlp_b2, cf_w1, cf_w2,
           cf_b2, blk_w, blk_b, out1_w, out1_b, out2_w, out2_b,
           ext1_w, ext1_b, ext2_w, ext2_b):
    starts = jnp.searchsorted(batch, jnp.arange(G + 1, dtype=batch.dtype))
    starts = starts.astype(jnp.int32)
    szf = (starts[1:] - starts[:-1]).astype(jnp.float32).reshape(G, 1, 1)
    idx = jnp.clip(starts[:-1, None] + jnp.arange(SMAX, dtype=jnp.int32)[None, :],
                   0, N - 1)
    posz = jnp.concatenate([pos, x.astype(jnp.float32)], axis=1)[idx]

    return _run(posz, szf, emb,
                mlp_w1, mlp_b1.reshape(T, 1, FIL),
                mlp_w2, mlp_b2.reshape(T, 1, FIL),
                cf_w1, cf_w2, cf_b2.reshape(T, 1, HID),
                blk_w, blk_b.reshape(T, 1, HID),
                out1_w, out1_b.reshape(1, HID // 2),
                out2_w, out2_b.reshape(1, 1),
                ext1_w, ext1_b.reshape(1, HID),
                ext2_w, ext2_b.reshape(1, NC))


# speedup vs baseline: 1.0415x; 1.0415x over previous
import functools

import jax
import jax.numpy as jnp
from jax.experimental import pallas as pl
from jax.experimental.pallas import tpu as pltpu

N = 10000
G = 64
HID = 128
FIL = 128
NG = 50
T = 3
CUT = 10.0
K = 32
NC = 10

SMAX = 256
CH = 8
NCH = K // CH

_LOG2 = 0.6931471805599453
_KC = 128.0
_STEP = CUT / (NG - 1)
_COEFF = -0.5 / (_STEP * _STEP)


def _fiota(shape, dim):
    return jax.lax.broadcasted_iota(jnp.int32, shape, dim).astype(jnp.float32)


def _dot(a, b):
    return jax.lax.dot_general(a, b, (((1,), (0,)), ((), ())),
                               preferred_element_type=jnp.float32)


def _ssp(v):
    return jnp.maximum(v, 0.0) + jnp.log(1.0 + jnp.exp(-jnp.abs(v))) - _LOG2


def _body(posz_ref, sz_ref, emb_ref, w1_ref, b1_ref, w2_ref, b2_ref,
          cfw1_ref, cfw2_ref, cfb2_ref, blkw_ref, blkb_ref,
          o1w_ref, o1b_ref, o2w_ref, o2b_ref,
          e1w_ref, e1b_ref, e2w_ref, e2b_ref,
          out_ref, acc_ref):
    f32 = jnp.float32
    g = pl.program_id(0)

    @pl.when(g == 0)
    def _():
        acc_ref[...] = jnp.zeros_like(acc_ref)

    pg = posz_ref[0]
    sz = sz_ref[0, 0, 0]
    posg = pg[:, 0:3]
    zg = pg[:, 3:4]

    eio = _fiota((SMAX, 100), 1)
    h = _dot((eio == zg).astype(f32), emb_ref[...])

    sq = jnp.sum(posg * posg, axis=1, keepdims=True)
    dpp = jax.lax.dot_general(posg, posg, (((1,), (1,)), ((), ())),
                              preferred_element_type=f32)
    sq_j = jax.lax.dot_general(jnp.ones((SMAX, 1), f32), sq,
                               (((1,), (1,)), ((), ())),
                               preferred_element_type=f32)
    d2 = jnp.maximum(sq + sq_j - 2.0 * dpp, 0.0)

    jio = _fiota((SMAX, SMAX), 1)
    iio = _fiota((SMAX, SMAX), 0)
    mask = (jio != iio) & (jio < sz) & (iio < sz)

    ji = jax.lax.broadcasted_iota(jnp.int32, (SMAX, SMAX), 1)
    key = jnp.maximum(_KC - d2, 0.0)
    ki = jax.lax.bitcast_convert_type(key, jnp.int32)
    ki = (ki & jnp.int32(-256)) | (jnp.int32(SMAX - 1) - ji)
    cur = jnp.where(mask, jax.lax.bitcast_convert_type(ki, f32), 0.0)

    m_l, ohk_l = [], []
    for _ in range(K):
        m = jnp.max(cur, axis=1, keepdims=True)
        sel = cur == m
        ohk_l.append((sel & (m > (_KC - CUT * CUT))).astype(f32))
        cur = jnp.where(sel, 0.0, cur)
        m_l.append(m)
    mm = jnp.concatenate(m_l, axis=1)
    valid = mm > (_KC - CUT * CUT)
    de = jnp.where(valid, jnp.maximum(_KC - mm, 0.0), 1.0)
    distm = jnp.sqrt(jnp.maximum(de, 1e-12))
    wctm = jnp.where(valid, 0.5 * (jnp.cos(distm * (jnp.pi / CUT)) + 1.0),
                     0.0)
    dist_l = [distm[:, k:k + 1] for k in range(K)]
    wct_l = [wctm[:, k:k + 1] for k in range(K)]

    offs = _fiota((1, NG), 1) * _STEP
    gs_l, oh_l, wc_l = [], [], []
    for c in range(NCH):
        ks = range(c * CH, (c + 1) * CH)
        gs_l.append(jnp.concatenate(
            [jnp.exp(_COEFF * (dist_l[k] - offs) ** 2) for k in ks], axis=0))
        oh_l.append(jnp.concatenate([ohk_l[k] for k in ks], axis=0))
        wc_l.append(jnp.concatenate([wct_l[k] for k in ks], axis=0))

    for t in range(T):
        w1 = w1_ref[t]
        b1 = b1_ref[t]
        w2 = w2_ref[t]
        b2 = b2_ref[t]
        xl = _dot(h, cfw1_ref[t])
        agg = jnp.zeros((SMAX, FIL), f32)
        for c in range(NCH):
            f1 = _ssp(_dot(gs_l[c], w1) + b1)
            wf = (_dot(f1, w2) + b2) * wc_l[c]
            xg = _dot(oh_l[c], xl)
            msg = xg * wf
            for kk in range(CH):
                agg = agg + msg[kk * SMAX:(kk + 1) * SMAX, :]
        hc = _dot(agg, cfw2_ref[t]) + cfb2_ref[t]
        h = h + _dot(_ssp(hc), blkw_ref[t]) + blkb_ref[t]

    h2 = _ssp(_dot(h, o1w_ref[...]) + o1b_ref[...])
    pa = _dot(h2, o2w_ref[...]) + o2b_ref[...]
    rv = (_fiota((SMAX, 1), 0) < sz).astype(f32)
    gsum = jnp.sum(pa * rv)
    gio = jax.lax.broadcasted_iota(jnp.int32, (G, 1), 0)
    acc_ref[...] = acc_ref[...] + jnp.where(gio == g, gsum, 0.0)

    @pl.when(g == G - 1)
    def _():
        accv = acc_ref[...]
        hid = jnp.maximum(accv * e1w_ref[...] + e1b_ref[...], 0.0)
        out_ref[...] = _dot(hid, e2w_ref[...]) + e2b_ref[...]


def _full(shape):
    nd = len(shape)
    return pl.BlockSpec(shape, lambda g, _nd=nd: (0,) * _nd)


@functools.partial(jax.jit, static_argnames=("interpret",))
def _run(posz, szf, emb, mlp_w1, mlp_b1, mlp_w2, mlp_b2, cf_w1, cf_w2, cf_b2,
         blk_w, blk_b, out1_w, out1_b, out2_w, out2_b,
         ext1_w, ext1_b, ext2_w, ext2_b, interpret=False):
    return pl.pallas_call(
        _body,
        grid=(G,),
        in_specs=[
            pl.BlockSpec((1, SMAX, 4), lambda g: (g, 0, 0)),
            pl.BlockSpec((1, 1, 1), lambda g: (g, 0, 0)),
            _full((100, HID)),
            _full((T, NG, FIL)), _full((T, 1, FIL)),
            _full((T, FIL, FIL)), _full((T, 1, FIL)),
            _full((T, HID, FIL)),
            _full((T, FIL, HID)), _full((T, 1, HID)),
            _full((T, HID, HID)), _full((T, 1, HID)),
            _full((HID, HID // 2)), _full((1, HID // 2)),
            _full((HID // 2, 1)), _full((1, 1)),
            _full((1, HID)), _full((1, HID)),
            _full((HID, NC)), _full((1, NC)),
        ],
        out_specs=pl.BlockSpec((G, NC), lambda g: (0, 0)),
        out_shape=jax.ShapeDtypeStruct((G, NC), jnp.float32),
        scratch_shapes=[pltpu.VMEM((G, 1), jnp.float32)],
        interpret=interpret,
    )(posz, szf, emb, mlp_w1, mlp_b1, mlp_w2, mlp_b2, cf_w1, cf_w2, cf_b2,
      blk_w, blk_b, out1_w, out1_b, out2_w, out2_b,
      ext1_w, ext1_b, ext2_w, ext2_b)


def kernel(x, pos, batch, emb, mlp_w1, mlp_b1, mlp_w2, mlp_b2, cf_w1, cf_w2,
           cf_b2, blk_w, blk_b, out1_w, out1_b, out2_w, out2_b,
           ext1_w, ext1_b, ext2_w, ext2_b):
    starts = jnp.searchsorted(batch, jnp.arange(G + 1, dtype=batch.dtype))
    starts = starts.astype(jnp.int32)
    szf = (starts[1:] - starts[:-1]).astype(jnp.float32).reshape(G, 1, 1)
    idx = jnp.clip(starts[:-1, None] + jnp.arange(SMAX, dtype=jnp.int32)[None, :],
                   0, N - 1)
    posz = jnp.concatenate([pos, x.astype(jnp.float32)], axis=1)[idx]

    return _run(posz, szf, emb,
                mlp_w1, mlp_b1.reshape(T, 1, FIL),
                mlp_w2, mlp_b2.reshape(T, 1, FIL),
                cf_w1, cf_w2, cf_b2.reshape(T, 1, HID),
                blk_w, blk_b.reshape(T, 1, HID),
                out1_w, out1_b.reshape(1, HID // 2),
                out2_w, out2_b.reshape(1, 1),
                ext1_w, ext1_b.reshape(1, HID),
                ext2_w, ext2_b.reshape(1, NC))


# speedup vs baseline: 1.0969x; 1.0531x over previous
import functools

import jax
import jax.numpy as jnp
from jax.experimental import pallas as pl
from jax.experimental.pallas import tpu as pltpu

N = 10000
G = 64
HID = 128
FIL = 128
NG = 50
T = 3
CUT = 10.0
K = 32
NC = 10

SMAX = 256
CH = 8
NCH = K // CH
NP = 2

_LOG2 = 0.6931471805599453
_KC = 128.0
_STEP = CUT / (NG - 1)
_COEFF = -0.5 / (_STEP * _STEP)


def _fiota(shape, dim):
    return jax.lax.broadcasted_iota(jnp.int32, shape, dim).astype(jnp.float32)


def _dot(a, b):
    return jax.lax.dot_general(a, b, (((1,), (0,)), ((), ())),
                               preferred_element_type=jnp.float32)


def _ssp(v):
    return jnp.maximum(v, 0.0) + jnp.log(1.0 + jnp.exp(-jnp.abs(v))) - _LOG2


def _body(posz_ref, sz_ref, emb_ref, w1_ref, b1_ref, w2_ref, b2_ref,
          cfw1_ref, cfw2_ref, cfb2_ref, blkw_ref, blkb_ref,
          o1w_ref, o1b_ref, o2w_ref, o2b_ref,
          e1w_ref, e1b_ref, e2w_ref, e2b_ref,
          out_ref, acc_ref):
    f32 = jnp.float32
    g = pl.program_id(0)

    @pl.when(g == 0)
    def _():
        acc_ref[...] = jnp.zeros_like(acc_ref)

    jio = _fiota((SMAX, SMAX), 1)
    iio = _fiota((SMAX, SMAX), 0)
    ji = jax.lax.broadcasted_iota(jnp.int32, (SMAX, SMAX), 1)
    eio = _fiota((SMAX, 100), 1)
    offs = _fiota((1, NG), 1) * _STEP

    sz_s, h_s, cur_s = [], [], []
    for s in range(NP):
        pg = posz_ref[s]
        sz = sz_ref[s, 0, 0]
        posg = pg[:, 0:3]
        zg = pg[:, 3:4]

        h = _dot((eio == zg).astype(f32), emb_ref[...])

        sq = jnp.sum(posg * posg, axis=1, keepdims=True)
        dpp = jax.lax.dot_general(posg, posg, (((1,), (1,)), ((), ())),
                                  preferred_element_type=f32)
        sq_j = jax.lax.dot_general(jnp.ones((SMAX, 1), f32), sq,
                                   (((1,), (1,)), ((), ())),
                                   preferred_element_type=f32)
        d2 = jnp.maximum(sq + sq_j - 2.0 * dpp, 0.0)
        mask = (jio != iio) & (jio < sz) & (iio < sz)

        key = jnp.maximum(_KC - d2, 0.0)
        ki = jax.lax.bitcast_convert_type(key, jnp.int32)
        ki = (ki & jnp.int32(-256)) | (jnp.int32(SMAX - 1) - ji)
        sz_s.append(sz)
        h_s.append(h)
        cur_s.append(jnp.where(mask, jax.lax.bitcast_convert_type(ki, f32),
                               0.0))

    m_ls = [[] for _ in range(NP)]
    ohk_ls = [[] for _ in range(NP)]
    for _ in range(K):
        for s in range(NP):
            cur = cur_s[s]
            m = jnp.max(cur, axis=1, keepdims=True)
            sel = cur == m
            ohk_ls[s].append((sel & (m > (_KC - CUT * CUT))).astype(f32))
            cur_s[s] = jnp.where(sel, 0.0, cur)
            m_ls[s].append(m)

    gs_s, oh_s, wc_s = [], [], []
    for s in range(NP):
        mm = jnp.concatenate(m_ls[s], axis=1)
        valid = mm > (_KC - CUT * CUT)
        de = jnp.where(valid, jnp.maximum(_KC - mm, 0.0), 1.0)
        distm = jnp.sqrt(jnp.maximum(de, 1e-12))
        wctm = jnp.where(valid,
                         0.5 * (jnp.cos(distm * (jnp.pi / CUT)) + 1.0), 0.0)
        gs_l, oh_l, wc_l = [], [], []
        for c in range(NCH):
            ks = range(c * CH, (c + 1) * CH)
            gs_l.append(jnp.concatenate(
                [jnp.exp(_COEFF * (distm[:, k:k + 1] - offs) ** 2)
                 for k in ks], axis=0))
            oh_l.append(jnp.concatenate([ohk_ls[s][k] for k in ks], axis=0))
            wc_l.append(jnp.concatenate(
                [wctm[:, k:k + 1] for k in ks], axis=0))
        gs_s.append(gs_l)
        oh_s.append(oh_l)
        wc_s.append(wc_l)

    for t in range(T):
        w1 = w1_ref[t]
        b1 = b1_ref[t]
        w2 = w2_ref[t]
        b2 = b2_ref[t]
        xl_s = [_dot(h_s[s], cfw1_ref[t]) for s in range(NP)]
        agg_s = [jnp.zeros((SMAX, FIL), f32) for _ in range(NP)]
        for c in range(NCH):
            for s in range(NP):
                f1 = _ssp(_dot(gs_s[s][c], w1) + b1)
                wf = (_dot(f1, w2) + b2) * wc_s[s][c]
                xg = _dot(oh_s[s][c], xl_s[s])
                msg = xg * wf
                agg = agg_s[s]
                for kk in range(CH):
                    agg = agg + msg[kk * SMAX:(kk + 1) * SMAX, :]
                agg_s[s] = agg
        for s in range(NP):
            hc = _dot(agg_s[s], cfw2_ref[t]) + cfb2_ref[t]
            h_s[s] = h_s[s] + _dot(_ssp(hc), blkw_ref[t]) + blkb_ref[t]

    gio = jax.lax.broadcasted_iota(jnp.int32, (G, 1), 0)
    upd = jnp.zeros((G, 1), f32)
    for s in range(NP):
        h2 = _ssp(_dot(h_s[s], o1w_ref[...]) + o1b_ref[...])
        pa = _dot(h2, o2w_ref[...]) + o2b_ref[...]
        rv = (_fiota((SMAX, 1), 0) < sz_s[s]).astype(f32)
        gsum = jnp.sum(pa * rv)
        upd = upd + jnp.where(gio == NP * g + s, gsum, 0.0)
    acc_ref[...] = acc_ref[...] + upd

    @pl.when(g == G // NP - 1)
    def _():
        accv = acc_ref[...]
        hid = jnp.maximum(accv * e1w_ref[...] + e1b_ref[...], 0.0)
        out_ref[...] = _dot(hid, e2w_ref[...]) + e2b_ref[...]


def _full(shape):
    nd = len(shape)
    return pl.BlockSpec(shape, lambda g, _nd=nd: (0,) * _nd)


@functools.partial(jax.jit, static_argnames=("interpret",))
def _run(posz, szf, emb, mlp_w1, mlp_b1, mlp_w2, mlp_b2, cf_w1, cf_w2, cf_b2,
         blk_w, blk_b, out1_w, out1_b, out2_w, out2_b,
         ext1_w, ext1_b, ext2_w, ext2_b, interpret=False):
    return pl.pallas_call(
        _body,
        grid=(G // NP,),
        in_specs=[
            pl.BlockSpec((NP, SMAX, 4), lambda g: (g, 0, 0)),
            pl.BlockSpec((NP, 1, 1), lambda g: (g, 0, 0)),
            _full((100, HID)),
            _full((T, NG, FIL)), _full((T, 1, FIL)),
            _full((T, FIL, FIL)), _full((T, 1, FIL)),
            _full((T, HID, FIL)),
            _full((T, FIL, HID)), _full((T, 1, HID)),
            _full((T, HID, HID)), _full((T, 1, HID)),
            _full((HID, HID // 2)), _full((1, HID // 2)),
            _full((HID // 2, 1)), _full((1, 1)),
            _full((1, HID)), _full((1, HID)),
            _full((HID, NC)), _full((1, NC)),
        ],
        out_specs=pl.BlockSpec((G, NC), lambda g: (0, 0)),
        out_shape=jax.ShapeDtypeStruct((G, NC), jnp.float32),
        scratch_shapes=[pltpu.VMEM((G, 1), jnp.float32)],
        interpret=interpret,
    )(posz, szf, emb, mlp_w1, mlp_b1, mlp_w2, mlp_b2, cf_w1, cf_w2, cf_b2,
      blk_w, blk_b, out1_w, out1_b, out2_w, out2_b,
      ext1_w, ext1_b, ext2_w, ext2_b)


def kernel(x, pos, batch, emb, mlp_w1, mlp_b1, mlp_w2, mlp_b2, cf_w1, cf_w2,
           cf_b2, blk_w, blk_b, out1_w, out1_b, out2_w, out2_b,
           ext1_w, ext1_b, ext2_w, ext2_b):
    starts = jnp.searchsorted(batch, jnp.arange(G + 1, dtype=batch.dtype))
    starts = starts.astype(jnp.int32)
    szf = (starts[1:] - starts[:-1]).astype(jnp.float32).reshape(G, 1, 1)
    idx = jnp.clip(starts[:-1, None] + jnp.arange(SMAX, dtype=jnp.int32)[None, :],
                   0, N - 1)
    posz = jnp.concatenate([pos, x.astype(jnp.float32)], axis=1)[idx]

    return _run(posz, szf, emb,
                mlp_w1, mlp_b1.reshape(T, 1, FIL),
                mlp_w2, mlp_b2.reshape(T, 1, FIL),
                cf_w1, cf_w2, cf_b2.reshape(T, 1, HID),
                blk_w, blk_b.reshape(T, 1, HID),
                out1_w, out1_b.reshape(1, HID // 2),
                out2_w, out2_b.reshape(1, 1),
                ext1_w, ext1_b.reshape(1, HID),
                ext2_w, ext2_b.reshape(1, NC))
